# baseline (speedup 1.0000x reference)
import functools

import jax
import jax.numpy as jnp
from jax import lax
from jax.experimental import pallas as pl
from jax.experimental.pallas import tpu as pltpu
from jax.experimental.pallas import tpu_sc as plsc

N, E, DH, DE = 10000, 320000, 128, 16
NC, NS = 2, 16
EPC = E // NC
EPT = EPC // NS
CH = 80
NCH = EPT // CH
GR = CH // 16
RPT = N // NS

_SMEM1 = pl.BlockSpec(memory_space=pltpu.SMEM)


def _pre_body(h_ref, W_ref, b_ref, as_ref, ad_ref, hs_ref, s_ref, d_ref, m_ref):
    hs = jnp.dot(h_ref[...], W_ref[...], preferred_element_type=jnp.float32)
    hs = hs + b_ref[...][None, :]
    hs_ref[...] = hs
    s = jnp.sum(hs * as_ref[...][None, :], axis=1)
    d = jnp.sum(hs * ad_ref[...][None, :], axis=1)
    s_ref[...] = s
    d_ref[...] = d
    m_ref[0] = jnp.max(s) + jnp.max(d)


def _dense_pre(h, W, b, a_s, a_d):
    return pl.pallas_call(
        _pre_body,
        out_shape=(
            jax.ShapeDtypeStruct((N, DH), jnp.float32),
            jax.ShapeDtypeStruct((N,), jnp.float32),
            jax.ShapeDtypeStruct((N,), jnp.float32),
            jax.ShapeDtypeStruct((1,), jnp.float32),
        ),
        out_specs=(pl.BlockSpec(), pl.BlockSpec(), pl.BlockSpec(), _SMEM1),
    )(h, W, b, a_s, a_d)


def _t_body(ea2_ref, We_ref, be_ref, ae_ref, t_ref, m_ref):
    ve = jnp.sum(We_ref[...] * ae_ref[...][None, :], axis=1)
    cst = jnp.sum(be_ref[...] * ae_ref[...])
    p_row = jax.lax.broadcasted_iota(jnp.int32, (16 * DH, DH), 0)
    p_col = jax.lax.broadcasted_iota(jnp.int32, (16 * DH, DH), 1)
    ve_rep = jnp.tile(ve, (DH,))
    VE = jnp.where(p_row // DE == p_col, ve_rep[:, None], 0.0)
    t2d = jnp.dot(ea2_ref[...], VE, preferred_element_type=jnp.float32) + cst
    t_ref[...] = t2d
    m_ref[0] = jnp.max(t2d)


def _edge_t(ea2, We, be, a_e):
    return pl.pallas_call(
        _t_body,
        out_shape=(
            jax.ShapeDtypeStruct((E // DH, DH), jnp.float32),
            jax.ShapeDtypeStruct((1,), jnp.float32),
        ),
        out_specs=(pl.BlockSpec(), _SMEM1),
    )(ea2, We, be, a_e)


def _post_body(G_ref, QD_ref, We_ref, be_ref, out_ref):
    Q = QD_ref[0, :, 0:16] + QD_ref[1, :, 0:16]
    Dd = QD_ref[0, :, 16:17] + QD_ref[1, :, 16:17]
    G = G_ref[0] + G_ref[1]
    acc = G + jnp.dot(Q, We_ref[...], preferred_element_type=jnp.float32)
    acc = acc + Dd * be_ref[...][None, :]
    acc = acc / (Dd + 1e-16)
    out_ref[...] = jnp.where(acc > 0, acc, jnp.exp(jnp.minimum(acc, 0.0)) - 1.0)


def _dense_post(G2, QD2, We, be):
    return pl.pallas_call(
        _post_body,
        out_shape=jax.ShapeDtypeStruct((N, DH), jnp.float32),
    )(G2, QD2, We, be)


_GD = lax.GatherDimensionNumbers(offset_dims=(), collapsed_slice_dims=(0,),
                                 start_index_map=(0,))


def _lane_splat(v, i):
    idx = jnp.full((16, 1), i, jnp.int32)
    return lax.gather(v, idx, _GD, (1,),
                      mode=lax.GatherScatterMode.PROMISE_IN_BOUNDS)

def _sc_body(sd_hbm, t_hbm, src_hbm, dst_hbm, ea_hbm, hs_hbm, mub_hbm,
             G_out, QD_out,
             G_acc, QD_acc, sd_sh, mub_v,
             srcA, dstA, tA, idxA, svdA, eaA, rowsA, qdA,
             srcB, dstB, tB, idxB, svdB, eaB, rowsB, qdB,
             sem_in, sem_g, sem_sc):
    c = lax.axis_index("c")
    sub = lax.axis_index("s")
    base = c * EPC + sub * EPT
    slab0 = sub * 640
    setA = (srcA, dstA, tA, idxA, svdA, eaA, rowsA, qdA)
    setB = (srcB, dstB, tB, idxB, svdB, eaB, rowsB, qdB)

    zero16 = jnp.zeros((16,), jnp.float32)
    for m in range(CH):
        for f in range(DH // 16):
            rowsA[m, pl.ds(16 * f, 16)] = zero16
        for f in range(2):
            qdA[m, pl.ds(16 * f, 16)] = zero16

    @pl.when(sub == 0)
    def _():
        pltpu.sync_copy(sd_hbm, sd_sh)

    @pl.when(sub < NS - 1)
    def _():
        for k in range(8):
            pltpu.sync_copy(rowsA, G_acc.at[pl.ds(slab0 + 80 * k, 80)])
            pltpu.sync_copy(qdA, QD_acc.at[pl.ds(slab0 + 80 * k, 80)])

    @pl.when(sub == NS - 1)
    def _():
        for k in range(5):
            pltpu.sync_copy(rowsA, G_acc.at[pl.ds(9600 + 80 * k, 80)])
            pltpu.sync_copy(qdA, QD_acc.at[pl.ds(9600 + 80 * k, 80)])

    plsc.subcore_barrier()

    pltpu.sync_copy(mub_hbm, mub_v)
    mub = mub_v[...]

    def _offsets(j):
        o = pl.multiple_of(base + j * CH, CH)
        oe = pl.multiple_of((base + j * CH) * DE, CH * DE)
        return o, oe

    def _fire_stage1(bufs, j):
        o, oe = _offsets(j)
        src_b, dst_b, t_b, _, _, ea_b, _, _ = bufs
        return (
            pltpu.async_copy(src_hbm.at[pl.ds(o, CH)], src_b, sem_in),
            pltpu.async_copy(dst_hbm.at[pl.ds(o, CH)], dst_b, sem_in),
            pltpu.async_copy(t_hbm.at[pl.ds(o, CH)], t_b, sem_in),
            pltpu.async_copy(ea_hbm.at[pl.ds(oe, CH * DE)], ea_b, sem_in),
        )

    def _do_stage2(bufs):
        src_b, dst_b, _, idx_b, svd_b, _, rows_b, _ = bufs
        hg = pltpu.async_copy(hs_hbm.at[src_b], rows_b, sem_g)
        for g in range(GR):
            idx_b[pl.ds(16 * g, 16)] = src_b[pl.ds(16 * g, 16)]
            idx_b[pl.ds(CH + 16 * g, 16)] = dst_b[pl.ds(16 * g, 16)] + N
        pltpu.sync_copy(sd_sh.at[idx_b], svd_b)
        hg.wait()

    def _compute(bufs):
        _, _, t_b, _, svd_b, ea_b, rows_b, qd_b = bufs
        for g in range(GR):
            sv = svd_b[pl.ds(16 * g, 16)]
            dv = svd_b[pl.ds(CH + 16 * g, 16)]
            tv = t_b[pl.ds(16 * g, 16)]
            z = sv + dv + tv
            l = jnp.where(z >= 0, z, 0.2 * z)
            ex = jnp.exp(l - mub)
            for i in range(16):
                m = 16 * g + i
                bs = _lane_splat(ex, i)
                qd_b[m, pl.ds(0, 16)] = ea_b[pl.ds(DE * m, 16)] * bs
                qd_b[m, pl.ds(16, 16)] = bs
        for m in range(CH):
            bs = qd_b[m, pl.ds(16, 16)]
            for f in range(DH // 16):
                rows_b[m, pl.ds(16 * f, 16)] = rows_b[m, pl.ds(16 * f, 16)] * bs

    def _fire_scatter(bufs):
        _, dst_b, _, _, _, _, rows_b, qd_b = bufs
        return (
            pltpu.async_copy(rows_b, G_acc.at[dst_b], sem_sc, add=True),
            pltpu.async_copy(qd_b, QD_acc.at[dst_b], sem_sc, add=True),
        )

    def _do_scatter(bufs):
        for h in _fire_scatter(bufs):
            h.wait()

    def _half(P, Q, j):
        jn = jnp.minimum(j + 1, NCH - 1)
        h1 = _fire_stage1(Q, jn)
        _compute(P)
        hsc = _fire_scatter(P)
        for h in h1:
            h.wait()
        _do_stage2(Q)
        for h in hsc:
            h.wait()

    for h in _fire_stage1(setA, 0):
        h.wait()
    _do_stage2(setA)

    def pair_body(i, carry):
        _half(setA, setB, 2 * i)
        _half(setB, setA, 2 * i + 1)
        return carry

    lax.fori_loop(0, (NCH - 1) // 2, pair_body, 0)

    _compute(setA)
    _do_scatter(setA)

    plsc.subcore_barrier()

    def _repack(r, carry):
        for k in range(4):
            for f in range(2):
                rowsA[r, pl.ds(32 * k + 16 * f, 16)] = \
                    qdA[4 * r + k, pl.ds(16 * f, 16)]
        return carry

    def _drain_qd(nblk, src0, dst0):
        for blk in range(nblk):
            pltpu.sync_copy(QD_acc.at[pl.ds(src0 + 80 * blk, 80)], qdA)
            lax.fori_loop(0, 20, _repack, 0)
            pltpu.sync_copy(rowsA.at[pl.ds(0, 20)],
                            QD_out.at[c, pl.ds(dst0 + 20 * blk, 20)])

    @pl.when(sub < NS - 1)
    def _():
        pltpu.sync_copy(G_acc.at[pl.ds(slab0, 640)],
                        G_out.at[c, pl.ds(slab0, 640)])
        _drain_qd(8, slab0, sub * 160)

    @pl.when(sub == NS - 1)
    def _():
        pltpu.sync_copy(G_acc.at[pl.ds(9600, 400)],
                        G_out.at[c, pl.ds(9600, 400)])
        _drain_qd(5, 9600, 2400)


def _sc_accumulate(sd, t, src, dst, ea_flat, hs, mub16):
    mesh = plsc.VectorSubcoreMesh(core_axis_name="c", subcore_axis_name="s")
    f = pl.kernel(
        _sc_body,
        mesh=mesh,
        compiler_params=pltpu.CompilerParams(needs_layout_passes=False,
                                             use_tc_tiling_on_sc=False),
        out_type=(
            jax.ShapeDtypeStruct((NC, N, DH), jnp.float32),
            jax.ShapeDtypeStruct((NC, N // 4, DH), jnp.float32),
        ),
        scratch_types=[
            pltpu.VMEM_SHARED((N, DH), jnp.float32),
            pltpu.VMEM_SHARED((N, 32), jnp.float32),
            pltpu.VMEM_SHARED((2 * N,), jnp.float32),
            pltpu.VMEM((16,), jnp.float32),
        ] + 2 * [
            pltpu.VMEM((CH,), jnp.int32),
            pltpu.VMEM((CH,), jnp.int32),
            pltpu.VMEM((CH,), jnp.float32),
            pltpu.VMEM((2 * CH,), jnp.int32),
            pltpu.VMEM((2 * CH,), jnp.float32),
            pltpu.VMEM((CH * DE,), jnp.float32),
            pltpu.VMEM((CH, DH), jnp.float32),
            pltpu.VMEM((CH, 32), jnp.float32),
        ] + [
            pltpu.SemaphoreType.DMA,
            pltpu.SemaphoreType.DMA,
            pltpu.SemaphoreType.DMA,
        ],
    )
    return f(sd, t, src, dst, ea_flat, hs, mub16)


def _layer(h, src, dst, ea_flat, ea2, W, b, We, be, a_s, a_d, a_e):
    hs, s, d, msd = _dense_pre(h, W, b, a_s, a_d)
    t2d, mt = _edge_t(ea2, We, be, a_e)
    t = t2d.reshape(E)
    zmax = msd[0] + mt[0]
    mub = jnp.where(zmax >= 0, zmax, 0.2 * zmax)
    mub16 = jnp.full((16,), mub, jnp.float32)
    G2, QDp = _sc_accumulate(jnp.concatenate([s, d]), t, src, dst,
                             ea_flat, hs, mub16)
    QD2 = QDp.reshape(NC, N, 32)
    return _dense_post(G2, QD2, We, be)


def kernel(x, edge_index, edge_attr, W0, b0, We0, be0, as0, ad0, ae0,
           W1, b1, We1, be1, as1, ad1, ae1):
    src = edge_index[0]
    dst = edge_index[1]
    ea_flat = edge_attr.reshape(E * DE)
    ea2 = ea_flat.reshape(E // DH, DH * DE)
    h = _layer(x, src, dst, ea_flat, ea2, W0, b0, We0, be0, as0, ad0, ae0)
    h = _layer(h, src, dst, ea_flat, ea2, W1, b1, We1, be1, as1, ad1, ae1)
    return h


# speedup vs baseline: 1.0020x; 1.0020x over previous
import functools

import jax
import jax.numpy as jnp
from jax import lax
from jax.experimental import pallas as pl
from jax.experimental.pallas import tpu as pltpu
from jax.experimental.pallas import tpu_sc as plsc

N, E, DH, DE = 10000, 320000, 128, 16
NC, NS = 2, 16
EPC = E // NC
EPT = EPC // NS
CH = 80
NCH = EPT // CH
GR = CH // 16
RPT = N // NS

_SMEM1 = pl.BlockSpec(memory_space=pltpu.SMEM)


def _pre_body(h_ref, W_ref, b_ref, as_ref, ad_ref, hs_ref, s_ref, d_ref, m_ref):
    hs = jnp.dot(h_ref[...], W_ref[...], preferred_element_type=jnp.float32)
    hs = hs + b_ref[...][None, :]
    hs_ref[...] = hs
    s = jnp.sum(hs * as_ref[...][None, :], axis=1)
    d = jnp.sum(hs * ad_ref[...][None, :], axis=1)
    s_ref[...] = s
    d_ref[...] = d
    m_ref[0] = jnp.max(s) + jnp.max(d)


def _dense_pre(h, W, b, a_s, a_d):
    return pl.pallas_call(
        _pre_body,
        out_shape=(
            jax.ShapeDtypeStruct((N, DH), jnp.float32),
            jax.ShapeDtypeStruct((N,), jnp.float32),
            jax.ShapeDtypeStruct((N,), jnp.float32),
            jax.ShapeDtypeStruct((1,), jnp.float32),
        ),
        out_specs=(pl.BlockSpec(), pl.BlockSpec(), pl.BlockSpec(), _SMEM1),
    )(h, W, b, a_s, a_d)


def _t_body(ea2_ref, We_ref, be_ref, ae_ref, t_ref, m_ref):
    ve = jnp.sum(We_ref[...] * ae_ref[...][None, :], axis=1)
    cst = jnp.sum(be_ref[...] * ae_ref[...])
    p_row = jax.lax.broadcasted_iota(jnp.int32, (16 * DH, DH), 0)
    p_col = jax.lax.broadcasted_iota(jnp.int32, (16 * DH, DH), 1)
    ve_rep = jnp.tile(ve, (DH,))
    VE = jnp.where(p_row // DE == p_col, ve_rep[:, None], 0.0)
    t2d = jnp.dot(ea2_ref[...], VE, preferred_element_type=jnp.float32) + cst
    t_ref[...] = t2d
    m_ref[0] = jnp.max(t2d)


def _edge_t(ea2, We, be, a_e):
    return pl.pallas_call(
        _t_body,
        out_shape=(
            jax.ShapeDtypeStruct((E // DH, DH), jnp.float32),
            jax.ShapeDtypeStruct((1,), jnp.float32),
        ),
        out_specs=(pl.BlockSpec(), _SMEM1),
    )(ea2, We, be, a_e)


def _post_body(G_ref, QD_ref, We_ref, be_ref, out_ref):
    Q = QD_ref[0, :, 0:16] + QD_ref[1, :, 0:16]
    Dd = QD_ref[0, :, 16:17] + QD_ref[1, :, 16:17]
    G = G_ref[0] + G_ref[1]
    acc = G + jnp.dot(Q, We_ref[...], preferred_element_type=jnp.float32)
    acc = acc + Dd * be_ref[...][None, :]
    acc = acc / (Dd + 1e-16)
    out_ref[...] = jnp.where(acc > 0, acc, jnp.exp(jnp.minimum(acc, 0.0)) - 1.0)


def _dense_post(G2, QD2, We, be):
    return pl.pallas_call(
        _post_body,
        out_shape=jax.ShapeDtypeStruct((N, DH), jnp.float32),
    )(G2, QD2, We, be)


_GD = lax.GatherDimensionNumbers(offset_dims=(), collapsed_slice_dims=(0,),
                                 start_index_map=(0,))


def _lane_splat(v, i):
    idx = jnp.full((16, 1), i, jnp.int32)
    return lax.gather(v, idx, _GD, (1,),
                      mode=lax.GatherScatterMode.PROMISE_IN_BOUNDS)

def _sc_body(sd_hbm, t_hbm, src_hbm, dst_hbm, ea_hbm, hs_hbm, mub_hbm,
             G_out, QD_out,
             G_acc, QD_acc, sd_sh, mub_v,
             srcA, dstA, tA, idxA, svdA, eaA, rowsA, qdA,
             srcB, dstB, tB, idxB, svdB, eaB, rowsB, qdB,
             sem_in, sem_g, sem_sv, sem_sc):
    c = lax.axis_index("c")
    sub = lax.axis_index("s")
    base = c * EPC + sub * EPT
    slab0 = sub * 640
    setA = (srcA, dstA, tA, idxA, svdA, eaA, rowsA, qdA)
    setB = (srcB, dstB, tB, idxB, svdB, eaB, rowsB, qdB)

    zero16 = jnp.zeros((16,), jnp.float32)
    for m in range(CH):
        for f in range(DH // 16):
            rowsA[m, pl.ds(16 * f, 16)] = zero16
        for f in range(2):
            qdA[m, pl.ds(16 * f, 16)] = zero16

    @pl.when(sub == 0)
    def _():
        pltpu.sync_copy(sd_hbm, sd_sh)

    @pl.when(sub < NS - 1)
    def _():
        for k in range(8):
            pltpu.sync_copy(rowsA, G_acc.at[pl.ds(slab0 + 80 * k, 80)])
            pltpu.sync_copy(qdA, QD_acc.at[pl.ds(slab0 + 80 * k, 80)])

    @pl.when(sub == NS - 1)
    def _():
        for k in range(5):
            pltpu.sync_copy(rowsA, G_acc.at[pl.ds(9600 + 80 * k, 80)])
            pltpu.sync_copy(qdA, QD_acc.at[pl.ds(9600 + 80 * k, 80)])

    plsc.subcore_barrier()

    pltpu.sync_copy(mub_hbm, mub_v)
    mub = mub_v[...]

    def _offsets(j):
        o = pl.multiple_of(base + j * CH, CH)
        oe = pl.multiple_of((base + j * CH) * DE, CH * DE)
        return o, oe

    def _fire_stage1(bufs, j):
        o, oe = _offsets(j)
        src_b, dst_b, t_b, _, _, ea_b, _, _ = bufs
        return (
            pltpu.async_copy(src_hbm.at[pl.ds(o, CH)], src_b, sem_in),
            pltpu.async_copy(dst_hbm.at[pl.ds(o, CH)], dst_b, sem_in),
            pltpu.async_copy(t_hbm.at[pl.ds(o, CH)], t_b, sem_in),
            pltpu.async_copy(ea_hbm.at[pl.ds(oe, CH * DE)], ea_b, sem_in),
        )

    def _fire_stage2(bufs):
        src_b, dst_b, _, idx_b, svd_b, _, rows_b, _ = bufs
        hg = pltpu.async_copy(hs_hbm.at[src_b], rows_b, sem_g)
        for g in range(GR):
            idx_b[pl.ds(16 * g, 16)] = src_b[pl.ds(16 * g, 16)]
            idx_b[pl.ds(CH + 16 * g, 16)] = dst_b[pl.ds(16 * g, 16)] + N
        hsv = pltpu.async_copy(sd_sh.at[idx_b], svd_b, sem_sv)
        return (hg, hsv)

    def _do_stage2(bufs):
        for h in _fire_stage2(bufs):
            h.wait()

    def _compute(bufs):
        _, _, t_b, _, svd_b, ea_b, rows_b, qd_b = bufs
        for g in range(GR):
            sv = svd_b[pl.ds(16 * g, 16)]
            dv = svd_b[pl.ds(CH + 16 * g, 16)]
            tv = t_b[pl.ds(16 * g, 16)]
            z = sv + dv + tv
            l = jnp.where(z >= 0, z, 0.2 * z)
            ex = jnp.exp(l - mub)
            for i in range(16):
                m = 16 * g + i
                bs = _lane_splat(ex, i)
                qd_b[m, pl.ds(0, 16)] = ea_b[pl.ds(DE * m, 16)] * bs
                qd_b[m, pl.ds(16, 16)] = bs
        for m in range(CH):
            bs = qd_b[m, pl.ds(16, 16)]
            for f in range(DH // 16):
                rows_b[m, pl.ds(16 * f, 16)] = rows_b[m, pl.ds(16 * f, 16)] * bs

    def _fire_scatter(bufs):
        _, dst_b, _, _, _, _, rows_b, qd_b = bufs
        return (
            pltpu.async_copy(rows_b, G_acc.at[dst_b], sem_sc, add=True),
            pltpu.async_copy(qd_b, QD_acc.at[dst_b], sem_sc, add=True),
        )

    def _do_scatter(bufs):
        for h in _fire_scatter(bufs):
            h.wait()

    def _half(P, Q, j):
        jn = jnp.minimum(j + 1, NCH - 1)
        h1 = _fire_stage1(Q, jn)
        _compute(P)
        hsc = _fire_scatter(P)
        for h in h1:
            h.wait()
        h2 = _fire_stage2(Q)
        for h in hsc:
            h.wait()
        for h in h2:
            h.wait()

    for h in _fire_stage1(setA, 0):
        h.wait()
    _do_stage2(setA)

    def pair_body(i, carry):
        _half(setA, setB, 2 * i)
        _half(setB, setA, 2 * i + 1)
        return carry

    lax.fori_loop(0, (NCH - 1) // 2, pair_body, 0)

    _compute(setA)
    _do_scatter(setA)

    plsc.subcore_barrier()

    def _repack(r, carry):
        for k in range(4):
            for f in range(2):
                rowsA[r, pl.ds(32 * k + 16 * f, 16)] = \
                    qdA[4 * r + k, pl.ds(16 * f, 16)]
        return carry

    def _drain_qd(nblk, src0, dst0):
        for blk in range(nblk):
            pltpu.sync_copy(QD_acc.at[pl.ds(src0 + 80 * blk, 80)], qdA)
            lax.fori_loop(0, 20, _repack, 0)
            pltpu.sync_copy(rowsA.at[pl.ds(0, 20)],
                            QD_out.at[c, pl.ds(dst0 + 20 * blk, 20)])

    @pl.when(sub < NS - 1)
    def _():
        pltpu.sync_copy(G_acc.at[pl.ds(slab0, 640)],
                        G_out.at[c, pl.ds(slab0, 640)])
        _drain_qd(8, slab0, sub * 160)

    @pl.when(sub == NS - 1)
    def _():
        pltpu.sync_copy(G_acc.at[pl.ds(9600, 400)],
                        G_out.at[c, pl.ds(9600, 400)])
        _drain_qd(5, 9600, 2400)


def _sc_accumulate(sd, t, src, dst, ea_flat, hs, mub16):
    mesh = plsc.VectorSubcoreMesh(core_axis_name="c", subcore_axis_name="s")
    f = pl.kernel(
        _sc_body,
        mesh=mesh,
        compiler_params=pltpu.CompilerParams(needs_layout_passes=False,
                                             use_tc_tiling_on_sc=False),
        out_type=(
            jax.ShapeDtypeStruct((NC, N, DH), jnp.float32),
            jax.ShapeDtypeStruct((NC, N // 4, DH), jnp.float32),
        ),
        scratch_types=[
            pltpu.VMEM_SHARED((N, DH), jnp.float32),
            pltpu.VMEM_SHARED((N, 32), jnp.float32),
            pltpu.VMEM_SHARED((2 * N,), jnp.float32),
            pltpu.VMEM((16,), jnp.float32),
        ] + 2 * [
            pltpu.VMEM((CH,), jnp.int32),
            pltpu.VMEM((CH,), jnp.int32),
            pltpu.VMEM((CH,), jnp.float32),
            pltpu.VMEM((2 * CH,), jnp.int32),
            pltpu.VMEM((2 * CH,), jnp.float32),
            pltpu.VMEM((CH * DE,), jnp.float32),
            pltpu.VMEM((CH, DH), jnp.float32),
            pltpu.VMEM((CH, 32), jnp.float32),
        ] + [
            pltpu.SemaphoreType.DMA,
            pltpu.SemaphoreType.DMA,
            pltpu.SemaphoreType.DMA,
            pltpu.SemaphoreType.DMA,
        ],
    )
    return f(sd, t, src, dst, ea_flat, hs, mub16)


def _layer(h, src, dst, ea_flat, ea2, W, b, We, be, a_s, a_d, a_e):
    hs, s, d, msd = _dense_pre(h, W, b, a_s, a_d)
    t2d, mt = _edge_t(ea2, We, be, a_e)
    t = t2d.reshape(E)
    zmax = msd[0] + mt[0]
    mub = jnp.where(zmax >= 0, zmax, 0.2 * zmax)
    mub16 = jnp.full((16,), mub, jnp.float32)
    G2, QDp = _sc_accumulate(jnp.concatenate([s, d]), t, src, dst,
                             ea_flat, hs, mub16)
    QD2 = QDp.reshape(NC, N, 32)
    return _dense_post(G2, QD2, We, be)


def kernel(x, edge_index, edge_attr, W0, b0, We0, be0, as0, ad0, ae0,
           W1, b1, We1, be1, as1, ad1, ae1):
    src = edge_index[0]
    dst = edge_index[1]
    ea_flat = edge_attr.reshape(E * DE)
    ea2 = ea_flat.reshape(E // DH, DH * DE)
    h = _layer(x, src, dst, ea_flat, ea2, W0, b0, We0, be0, as0, ad0, ae0)
    h = _layer(h, src, dst, ea_flat, ea2, W1, b1, We1, be1, as1, ad1, ae1)
    return h


# speedup vs baseline: 1.0063x; 1.0044x over previous
import functools

import jax
import jax.numpy as jnp
from jax import lax
from jax.experimental import pallas as pl
from jax.experimental.pallas import tpu as pltpu
from jax.experimental.pallas import tpu_sc as plsc

N, E, DH, DE = 10000, 320000, 128, 16
NC, NS = 2, 16
EPC = E // NC
EPT = EPC // NS
CH = 80
NCH = EPT // CH
GR = CH // 16
RPT = N // NS

_SMEM1 = pl.BlockSpec(memory_space=pltpu.SMEM)


def _emit_pre(h, W_ref, b_ref, as_ref, ad_ref, hs_ref, sd_ref, m_ref):
    hs = jnp.dot(h, W_ref[...], preferred_element_type=jnp.float32)
    hs = hs + b_ref[...][None, :]
    hs_ref[...] = hs
    s = jnp.sum(hs * as_ref[...][None, :], axis=1)
    d = jnp.sum(hs * ad_ref[...][None, :], axis=1)
    sd_ref[pl.ds(0, N)] = s
    sd_ref[pl.ds(N, N)] = d
    m_ref[0] = jnp.max(s) + jnp.max(d)


def _pre_body(h_ref, W_ref, b_ref, as_ref, ad_ref, hs_ref, sd_ref, m_ref):
    _emit_pre(h_ref[...], W_ref, b_ref, as_ref, ad_ref, hs_ref, sd_ref, m_ref)


_PRE_OUT = (
    jax.ShapeDtypeStruct((N, DH), jnp.float32),
    jax.ShapeDtypeStruct((2 * N,), jnp.float32),
    jax.ShapeDtypeStruct((1,), jnp.float32),
)


def _dense_pre(h, W, b, a_s, a_d):
    return pl.pallas_call(
        _pre_body,
        out_shape=_PRE_OUT,
        out_specs=(pl.BlockSpec(), pl.BlockSpec(), _SMEM1),
    )(h, W, b, a_s, a_d)


def _t_one(ea2, We_ref, be_ref, ae_ref, t_ref, m_ref):
    ve = jnp.sum(We_ref[...] * ae_ref[...][None, :], axis=1)
    cst = jnp.sum(be_ref[...] * ae_ref[...])
    p_row = jax.lax.broadcasted_iota(jnp.int32, (16 * DH, DH), 0)
    p_col = jax.lax.broadcasted_iota(jnp.int32, (16 * DH, DH), 1)
    ve_rep = jnp.tile(ve, (DH,))
    VE = jnp.where(p_row // DE == p_col, ve_rep[:, None], 0.0)
    t2d = jnp.dot(ea2, VE, preferred_element_type=jnp.float32) + cst
    t_ref[...] = t2d
    m_ref[0] = jnp.max(t2d)


def _t_body(ea2_ref, We0_ref, be0_ref, ae0_ref, We1_ref, be1_ref, ae1_ref,
            t0_ref, m0_ref, t1_ref, m1_ref):
    ea2 = ea2_ref[...]
    _t_one(ea2, We0_ref, be0_ref, ae0_ref, t0_ref, m0_ref)
    _t_one(ea2, We1_ref, be1_ref, ae1_ref, t1_ref, m1_ref)


def _edge_t2(ea2, We0, be0, ae0, We1, be1, ae1):
    return pl.pallas_call(
        _t_body,
        out_shape=(
            jax.ShapeDtypeStruct((E // DH, DH), jnp.float32),
            jax.ShapeDtypeStruct((1,), jnp.float32),
            jax.ShapeDtypeStruct((E // DH, DH), jnp.float32),
            jax.ShapeDtypeStruct((1,), jnp.float32),
        ),
        out_specs=(pl.BlockSpec(), _SMEM1, pl.BlockSpec(), _SMEM1),
    )(ea2, We0, be0, ae0, We1, be1, ae1)


def _merge_post(G_ref, QD_ref, We_ref, be_ref):
    Q = QD_ref[0, :, 0:16] + QD_ref[1, :, 0:16]
    Dd = QD_ref[0, :, 16:17] + QD_ref[1, :, 16:17]
    G = G_ref[0] + G_ref[1]
    acc = G + jnp.dot(Q, We_ref[...], preferred_element_type=jnp.float32)
    acc = acc + Dd * be_ref[...][None, :]
    acc = acc / (Dd + 1e-16)
    return jnp.where(acc > 0, acc, jnp.exp(jnp.minimum(acc, 0.0)) - 1.0)


def _post_body(G_ref, QD_ref, We_ref, be_ref, out_ref):
    out_ref[...] = _merge_post(G_ref, QD_ref, We_ref, be_ref)


def _dense_post(G2, QD2, We, be):
    return pl.pallas_call(
        _post_body,
        out_shape=jax.ShapeDtypeStruct((N, DH), jnp.float32),
    )(G2, QD2, We, be)


def _mid_body(G_ref, QD_ref, We0_ref, be0_ref, W1_ref, b1_ref,
              as1_ref, ad1_ref, hs_ref, sd_ref, m_ref):
    h = _merge_post(G_ref, QD_ref, We0_ref, be0_ref)
    _emit_pre(h, W1_ref, b1_ref, as1_ref, ad1_ref, hs_ref, sd_ref, m_ref)


def _dense_mid(G2, QD2, We0, be0, W1, b1, as1, ad1):
    return pl.pallas_call(
        _mid_body,
        out_shape=_PRE_OUT,
        out_specs=(pl.BlockSpec(), pl.BlockSpec(), _SMEM1),
    )(G2, QD2, We0, be0, W1, b1, as1, ad1)


_GD = lax.GatherDimensionNumbers(offset_dims=(), collapsed_slice_dims=(0,),
                                 start_index_map=(0,))


def _lane_splat(v, i):
    idx = jnp.full((16, 1), i, jnp.int32)
    return lax.gather(v, idx, _GD, (1,),
                      mode=lax.GatherScatterMode.PROMISE_IN_BOUNDS)

def _sc_body(sd_hbm, t_hbm, src_hbm, dst_hbm, ea_hbm, hs_hbm, mub_hbm,
             G_out, QD_out,
             G_acc, QD_acc, sd_sh, mub_v,
             srcA, dstA, tA, idxA, svdA, eaA, rowsA, qdA,
             srcB, dstB, tB, idxB, svdB, eaB, rowsB, qdB,
             sem_in, sem_g, sem_sv, sem_sc):
    c = lax.axis_index("c")
    sub = lax.axis_index("s")
    base = c * EPC + sub * EPT
    slab0 = sub * 640
    setA = (srcA, dstA, tA, idxA, svdA, eaA, rowsA, qdA)
    setB = (srcB, dstB, tB, idxB, svdB, eaB, rowsB, qdB)

    zero16 = jnp.zeros((16,), jnp.float32)
    for m in range(CH):
        for f in range(DH // 16):
            rowsA[m, pl.ds(16 * f, 16)] = zero16
        for f in range(2):
            qdA[m, pl.ds(16 * f, 16)] = zero16

    @pl.when(sub == 0)
    def _():
        pltpu.sync_copy(sd_hbm, sd_sh)

    @pl.when(sub < NS - 1)
    def _():
        for k in range(8):
            pltpu.sync_copy(rowsA, G_acc.at[pl.ds(slab0 + 80 * k, 80)])
            pltpu.sync_copy(qdA, QD_acc.at[pl.ds(slab0 + 80 * k, 80)])

    @pl.when(sub == NS - 1)
    def _():
        for k in range(5):
            pltpu.sync_copy(rowsA, G_acc.at[pl.ds(9600 + 80 * k, 80)])
            pltpu.sync_copy(qdA, QD_acc.at[pl.ds(9600 + 80 * k, 80)])

    plsc.subcore_barrier()

    pltpu.sync_copy(mub_hbm, mub_v)
    mub = mub_v[...]

    def _offsets(j):
        o = pl.multiple_of(base + j * CH, CH)
        oe = pl.multiple_of((base + j * CH) * DE, CH * DE)
        return o, oe

    def _fire_stage1(bufs, j):
        o, oe = _offsets(j)
        src_b, dst_b, t_b, _, _, ea_b, _, _ = bufs
        return (
            pltpu.async_copy(src_hbm.at[pl.ds(o, CH)], src_b, sem_in),
            pltpu.async_copy(dst_hbm.at[pl.ds(o, CH)], dst_b, sem_in),
            pltpu.async_copy(t_hbm.at[pl.ds(o, CH)], t_b, sem_in),
            pltpu.async_copy(ea_hbm.at[pl.ds(oe, CH * DE)], ea_b, sem_in),
        )

    def _fire_stage2(bufs):
        src_b, dst_b, _, idx_b, svd_b, _, rows_b, _ = bufs
        hg = pltpu.async_copy(hs_hbm.at[src_b], rows_b, sem_g)
        for g in range(GR):
            idx_b[pl.ds(16 * g, 16)] = src_b[pl.ds(16 * g, 16)]
            idx_b[pl.ds(CH + 16 * g, 16)] = dst_b[pl.ds(16 * g, 16)] + N
        hsv = pltpu.async_copy(sd_sh.at[idx_b], svd_b, sem_sv)
        return (hg, hsv)

    def _do_stage2(bufs):
        for h in _fire_stage2(bufs):
            h.wait()

    def _compute(bufs):
        _, _, t_b, _, svd_b, ea_b, rows_b, qd_b = bufs
        for g in range(GR):
            sv = svd_b[pl.ds(16 * g, 16)]
            dv = svd_b[pl.ds(CH + 16 * g, 16)]
            tv = t_b[pl.ds(16 * g, 16)]
            z = sv + dv + tv
            l = jnp.where(z >= 0, z, 0.2 * z)
            ex = jnp.exp(l - mub)
            for i in range(16):
                m = 16 * g + i
                bs = _lane_splat(ex, i)
                qd_b[m, pl.ds(0, 16)] = ea_b[pl.ds(DE * m, 16)] * bs
                qd_b[m, pl.ds(16, 16)] = bs
        for m in range(CH):
            bs = qd_b[m, pl.ds(16, 16)]
            for f in range(DH // 16):
                rows_b[m, pl.ds(16 * f, 16)] = rows_b[m, pl.ds(16 * f, 16)] * bs

    def _fire_scatter(bufs):
        _, dst_b, _, _, _, _, rows_b, qd_b = bufs
        return (
            pltpu.async_copy(rows_b, G_acc.at[dst_b], sem_sc, add=True),
            pltpu.async_copy(qd_b, QD_acc.at[dst_b], sem_sc, add=True),
        )

    def _do_scatter(bufs):
        for h in _fire_scatter(bufs):
            h.wait()

    def _half(P, Q, j):
        jn = jnp.minimum(j + 1, NCH - 1)
        h1 = _fire_stage1(Q, jn)
        _compute(P)
        hsc = _fire_scatter(P)
        for h in h1:
            h.wait()
        h2 = _fire_stage2(Q)
        for h in hsc:
            h.wait()
        for h in h2:
            h.wait()

    for h in _fire_stage1(setA, 0):
        h.wait()
    _do_stage2(setA)

    def pair_body(i, carry):
        _half(setA, setB, 2 * i)
        _half(setB, setA, 2 * i + 1)
        return carry

    lax.fori_loop(0, (NCH - 1) // 2, pair_body, 0)

    _compute(setA)
    _do_scatter(setA)

    plsc.subcore_barrier()

    def _repack(r, carry):
        for k in range(4):
            for f in range(2):
                rowsA[r, pl.ds(32 * k + 16 * f, 16)] = \
                    qdA[4 * r + k, pl.ds(16 * f, 16)]
        return carry

    def _drain_qd(nblk, src0, dst0):
        for blk in range(nblk):
            pltpu.sync_copy(QD_acc.at[pl.ds(src0 + 80 * blk, 80)], qdA)
            lax.fori_loop(0, 20, _repack, 0)
            pltpu.sync_copy(rowsA.at[pl.ds(0, 20)],
                            QD_out.at[c, pl.ds(dst0 + 20 * blk, 20)])

    @pl.when(sub < NS - 1)
    def _():
        pltpu.sync_copy(G_acc.at[pl.ds(slab0, 640)],
                        G_out.at[c, pl.ds(slab0, 640)])
        _drain_qd(8, slab0, sub * 160)

    @pl.when(sub == NS - 1)
    def _():
        pltpu.sync_copy(G_acc.at[pl.ds(9600, 400)],
                        G_out.at[c, pl.ds(9600, 400)])
        _drain_qd(5, 9600, 2400)


def _sc_accumulate(sd, t, src, dst, ea_flat, hs, mub16):
    mesh = plsc.VectorSubcoreMesh(core_axis_name="c", subcore_axis_name="s")
    f = pl.kernel(
        _sc_body,
        mesh=mesh,
        compiler_params=pltpu.CompilerParams(needs_layout_passes=False,
                                             use_tc_tiling_on_sc=False),
        out_type=(
            jax.ShapeDtypeStruct((NC, N, DH), jnp.float32),
            jax.ShapeDtypeStruct((NC, N // 4, DH), jnp.float32),
        ),
        scratch_types=[
            pltpu.VMEM_SHARED((N, DH), jnp.float32),
            pltpu.VMEM_SHARED((N, 32), jnp.float32),
            pltpu.VMEM_SHARED((2 * N,), jnp.float32),
            pltpu.VMEM((16,), jnp.float32),
        ] + 2 * [
            pltpu.VMEM((CH,), jnp.int32),
            pltpu.VMEM((CH,), jnp.int32),
            pltpu.VMEM((CH,), jnp.float32),
            pltpu.VMEM((2 * CH,), jnp.int32),
            pltpu.VMEM((2 * CH,), jnp.float32),
            pltpu.VMEM((CH * DE,), jnp.float32),
            pltpu.VMEM((CH, DH), jnp.float32),
            pltpu.VMEM((CH, 32), jnp.float32),
        ] + [
            pltpu.SemaphoreType.DMA,
            pltpu.SemaphoreType.DMA,
            pltpu.SemaphoreType.DMA,
            pltpu.SemaphoreType.DMA,
        ],
    )
    return f(sd, t, src, dst, ea_flat, hs, mub16)


def _mub16(msd, mt):
    zmax = msd[0] + mt[0]
    mub = jnp.where(zmax >= 0, zmax, 0.2 * zmax)
    return jnp.full((16,), mub, jnp.float32)


def kernel(x, edge_index, edge_attr, W0, b0, We0, be0, as0, ad0, ae0,
           W1, b1, We1, be1, as1, ad1, ae1):
    src = edge_index[0]
    dst = edge_index[1]
    ea_flat = edge_attr.reshape(E * DE)
    ea2 = ea_flat.reshape(E // DH, DH * DE)
    t2d0, mt0, t2d1, mt1 = _edge_t2(ea2, We0, be0, ae0, We1, be1, ae1)
    hs0, sd0, m0 = _dense_pre(x, W0, b0, as0, ad0)
    G2, QDp = _sc_accumulate(sd0, t2d0.reshape(E), src, dst, ea_flat, hs0,
                             _mub16(m0, mt0))
    hs1, sd1, m1 = _dense_mid(G2, QDp.reshape(NC, N, 32), We0, be0,
                              W1, b1, as1, ad1)
    G2, QDp = _sc_accumulate(sd1, t2d1.reshape(E), src, dst, ea_flat, hs1,
                             _mub16(m1, mt1))
    return _dense_post(G2, QDp.reshape(NC, N, 32), We1, be1)
